# baseline (speedup 1.0000x reference)
import functools

import jax
import jax.numpy as jnp
from jax import lax
from jax.experimental import pallas as pl
from jax.experimental.pallas import tpu as pltpu
from jax.experimental.pallas import tpu_sc as plsc

N = 10000
NP = 10240
D = 128
NCLS = 40
E = 320000
C = 56
NW = 32
KPT = -(-E // (NW * C))
EP = NW * C * KPT
RPS = NP // 16

_mesh = plsc.VectorSubcoreMesh(core_axis_name="c", subcore_axis_name="s")


@functools.partial(
    pl.kernel,
    out_type=jax.ShapeDtypeStruct((2, NP, D), jnp.float32),
    mesh=_mesh,
    scratch_types=[
        pltpu.VMEM((KPT, C), jnp.int32),
        pltpu.VMEM((C, D), jnp.float32),
        pltpu.VMEM_SHARED((NP, D), jnp.float32),
    ],
)
def _sc_degree(dsts_hbm, ones_hbm, zeros_hbm, out_hbm, dstv, onesv, dsh):
    c = lax.axis_index("c")
    s = lax.axis_index("s")
    w = c * 16 + s
    pltpu.sync_copy(dsts_hbm.at[w], dstv)
    pltpu.sync_copy(ones_hbm, onesv)
    pltpu.sync_copy(zeros_hbm.at[pl.ds(s * RPS, RPS)], dsh.at[pl.ds(s * RPS, RPS)])
    plsc.subcore_barrier()

    def body(j, carry):
        pltpu.sync_copy(onesv, dsh.at[dstv.at[j]], add=True)
        return carry

    lax.fori_loop(0, KPT, body, 0)
    plsc.subcore_barrier()
    pltpu.sync_copy(dsh.at[pl.ds(s * RPS, RPS)], out_hbm.at[c, pl.ds(s * RPS, RPS)])


@functools.partial(
    pl.kernel,
    out_type=jax.ShapeDtypeStruct((2, NP, D), jnp.float32),
    mesh=_mesh,
    scratch_types=[
        pltpu.VMEM((7, C), jnp.int32),
        pltpu.VMEM((7, C), jnp.int32),
        pltpu.VMEM((6, C, D), jnp.float32),
        pltpu.VMEM_SHARED((NP, D), jnp.float32),
        pltpu.SemaphoreType.DMA,
        pltpu.SemaphoreType.DMA,
        pltpu.SemaphoreType.DMA,
    ],
)
def _sc_aggregate(g_hbm, srcs_hbm, dsts_hbm, zeros_hbm, out_hbm,
                  srcv, dstv, buf, ysh, gsem, ssem, dsem):
    c = lax.axis_index("c")
    s = lax.axis_index("s")
    w = c * 16 + s
    pltpu.sync_copy(zeros_hbm.at[pl.ds(s * RPS, RPS)], ysh.at[pl.ds(s * RPS, RPS)])
    for b in range(6):
        pltpu.async_copy(srcs_hbm.at[w, b], srcv.at[b], ssem)
        pltpu.async_copy(dsts_hbm.at[w, b], dstv.at[b], dsem)
    plsc.subcore_barrier()

    for b in range(5):
        pltpu.make_async_copy(srcs_hbm.at[w, 0], srcv.at[b], ssem).wait()
        pltpu.async_copy(g_hbm.at[srcv.at[b]], buf.at[b], gsem)

    def body(j, carry):
        @pl.when(j + 6 < KPT)
        def _():
            r6 = lax.rem(j + 6, 7)
            pltpu.async_copy(srcs_hbm.at[w, j + 6], srcv.at[r6], ssem)
            pltpu.async_copy(dsts_hbm.at[w, j + 6], dstv.at[r6], dsem)

        @pl.when(j + 5 < KPT)
        def _():
            r5 = lax.rem(j + 5, 7)
            pltpu.make_async_copy(srcs_hbm.at[w, 0], srcv.at[r5], ssem).wait()
            pltpu.async_copy(g_hbm.at[srcv.at[r5]], buf.at[lax.rem(j + 5, 6)], gsem)

        p = lax.rem(j, 6)
        pltpu.make_async_copy(g_hbm.at[srcv.at[0]], buf.at[p], gsem).wait()
        pltpu.make_async_copy(dsts_hbm.at[w, 0], dstv.at[0], dsem).wait()
        pltpu.sync_copy(buf.at[p], ysh.at[dstv.at[lax.rem(j, 7)]], add=True)
        return carry

    lax.fori_loop(0, KPT, body, 0)
    plsc.subcore_barrier()
    pltpu.sync_copy(ysh.at[pl.ds(s * RPS, RPS)], out_hbm.at[c, pl.ds(s * RPS, RPS)])


_BLK = 512


def _prep_body(d2_ref, x_ref, dv_ref, g_ref):
    deg = d2_ref[0][:, 0:1] + d2_ref[1][:, 0:1] + 1.0
    dinv = lax.rsqrt(deg)
    dvb = jnp.broadcast_to(dinv, x_ref.shape)
    dv_ref[...] = dvb
    g_ref[...] = dvb * x_ref[...]


def _tc_prep(deg2, xp):
    grid = (NP // _BLK,)
    return pl.pallas_call(
        _prep_body,
        grid=grid,
        in_specs=[
            pl.BlockSpec((2, _BLK, D), lambda i: (0, i, 0)),
            pl.BlockSpec((_BLK, D), lambda i: (i, 0)),
        ],
        out_specs=[
            pl.BlockSpec((_BLK, D), lambda i: (i, 0)),
            pl.BlockSpec((_BLK, D), lambda i: (i, 0)),
        ],
        out_shape=[
            jax.ShapeDtypeStruct((NP, D), jnp.float32),
            jax.ShapeDtypeStruct((NP, D), jnp.float32),
        ],
    )(deg2, xp)


def _layer_body(relu, want_g, a_ref, g_ref, dv_ref, w_ref, b_ref, *out_refs):
    z = dv_ref[...] * (a_ref[0] + a_ref[1] + g_ref[...])
    h = jnp.dot(z, w_ref[...], preferred_element_type=jnp.float32,
                precision=lax.Precision.HIGHEST) + b_ref[...]
    if relu:
        h = jnp.maximum(h, 0.0)
    out_refs[0][...] = h
    if want_g:
        out_refs[1][...] = dv_ref[...] * h


def _tc_layer(a2, g, dvb, W, b, relu, want_g):
    F = W.shape[1]
    grid = (NP // _BLK,)
    out_shape = [jax.ShapeDtypeStruct((NP, F), jnp.float32)]
    out_specs = [pl.BlockSpec((_BLK, F), lambda i: (i, 0))]
    if want_g:
        out_shape.append(jax.ShapeDtypeStruct((NP, F), jnp.float32))
        out_specs.append(pl.BlockSpec((_BLK, F), lambda i: (i, 0)))
    return pl.pallas_call(
        functools.partial(_layer_body, relu, want_g),
        grid=grid,
        in_specs=[
            pl.BlockSpec((2, _BLK, D), lambda i: (0, i, 0)),
            pl.BlockSpec((_BLK, D), lambda i: (i, 0)),
            pl.BlockSpec((_BLK, D), lambda i: (i, 0)),
            pl.BlockSpec((D, F), lambda i: (0, 0)),
            pl.BlockSpec((1, F), lambda i: (0, 0)),
        ],
        out_specs=out_specs,
        out_shape=out_shape,
    )(a2, g, dvb, W, b)


def kernel(x, edge_index, W1, b1, W2, b2, W3, b3):
    src = edge_index[0].astype(jnp.int32)
    dst = edge_index[1].astype(jnp.int32)
    pad = EP - E
    src = jnp.concatenate([src, jnp.zeros((pad,), jnp.int32)])
    dst = jnp.concatenate([dst, jnp.full((pad,), N, jnp.int32)])
    srcs = src.reshape(NW, KPT, C)
    dsts = dst.reshape(NW, KPT, C)

    xp = jnp.pad(x, ((0, NP - N), (0, 0)))
    onesD = jnp.ones((C, D), jnp.float32)
    zerosD = jnp.zeros((NP, D), jnp.float32)

    deg2 = _sc_degree(dsts, onesD, zerosD)
    dvb, g1 = _tc_prep(deg2, xp)

    a1 = _sc_aggregate(g1, srcs, dsts, zerosD)
    h1, g2 = _tc_layer(a1, g1, dvb, W1, b1.reshape(1, -1), True, True)

    a2 = _sc_aggregate(g2, srcs, dsts, zerosD)
    h2, g3 = _tc_layer(a2, g2, dvb, W2, b2.reshape(1, -1), True, True)

    a3 = _sc_aggregate(g3, srcs, dsts, zerosD)
    (out,) = _tc_layer(a3, g3, dvb, W3, b3.reshape(1, -1), False, False)

    return (h2[:N], out[:N])


# speedup vs baseline: 1.1872x; 1.1872x over previous
import functools

import jax
import jax.numpy as jnp
from jax import lax
from jax.experimental import pallas as pl
from jax.experimental.pallas import tpu as pltpu
from jax.experimental.pallas import tpu_sc as plsc

N = 10000
NP = 10240
D = 128
NCLS = 40
E = 320000
C = 72
NW = 32
KPT = -(-E // (NW * C))
EP = NW * C * KPT
RPS = NP // 16

_mesh = plsc.VectorSubcoreMesh(core_axis_name="c", subcore_axis_name="s")


@functools.partial(
    pl.kernel,
    out_type=jax.ShapeDtypeStruct((2, NP, D), jnp.float32),
    mesh=_mesh,
    scratch_types=[
        pltpu.VMEM((KPT, C), jnp.int32),
        pltpu.VMEM((C, D), jnp.float32),
        pltpu.VMEM_SHARED((NP, D), jnp.float32),
    ],
)
def _sc_degree(dsts_hbm, ones_hbm, zeros_hbm, out_hbm, dstv, onesv, dsh):
    c = lax.axis_index("c")
    s = lax.axis_index("s")
    w = c * 16 + s
    pltpu.sync_copy(dsts_hbm.at[w], dstv)
    pltpu.sync_copy(ones_hbm, onesv)
    pltpu.sync_copy(zeros_hbm.at[pl.ds(s * RPS, RPS)], dsh.at[pl.ds(s * RPS, RPS)])
    plsc.subcore_barrier()

    def body(j, carry):
        pltpu.sync_copy(onesv, dsh.at[dstv.at[j]], add=True)
        return carry

    lax.fori_loop(0, KPT, body, 0)
    plsc.subcore_barrier()
    pltpu.sync_copy(dsh.at[pl.ds(s * RPS, RPS)], out_hbm.at[c, pl.ds(s * RPS, RPS)])


@functools.partial(
    pl.kernel,
    out_type=jax.ShapeDtypeStruct((2, NP, D), jnp.float32),
    mesh=_mesh,
    scratch_types=[
        pltpu.VMEM((6, C), jnp.int32),
        pltpu.VMEM((6, C), jnp.int32),
        pltpu.VMEM((5, C, D), jnp.float32),
        pltpu.VMEM_SHARED((NP, D), jnp.float32),
        pltpu.SemaphoreType.DMA,
        pltpu.SemaphoreType.DMA,
        pltpu.SemaphoreType.DMA,
    ],
)
def _sc_aggregate(g_hbm, srcs_hbm, dsts_hbm, zeros_hbm, out_hbm,
                  srcv, dstv, buf, ysh, gsem, ssem, dsem):
    c = lax.axis_index("c")
    s = lax.axis_index("s")
    w = c * 16 + s
    pltpu.sync_copy(zeros_hbm.at[pl.ds(s * RPS, RPS)], ysh.at[pl.ds(s * RPS, RPS)])
    for b in range(5):
        pltpu.async_copy(srcs_hbm.at[w, b], srcv.at[b], ssem)
        pltpu.async_copy(dsts_hbm.at[w, b], dstv.at[b], dsem)
    plsc.subcore_barrier()

    for b in range(4):
        pltpu.make_async_copy(srcs_hbm.at[w, 0], srcv.at[b], ssem).wait()
        pltpu.async_copy(g_hbm.at[srcv.at[b]], buf.at[b], gsem)

    def body(j, carry):
        @pl.when(j + 5 < KPT)
        def _():
            r5 = lax.rem(j + 5, 6)
            pltpu.async_copy(srcs_hbm.at[w, j + 5], srcv.at[r5], ssem)
            pltpu.async_copy(dsts_hbm.at[w, j + 5], dstv.at[r5], dsem)

        @pl.when(j + 4 < KPT)
        def _():
            r4 = lax.rem(j + 4, 6)
            pltpu.make_async_copy(srcs_hbm.at[w, 0], srcv.at[r4], ssem).wait()
            pltpu.async_copy(g_hbm.at[srcv.at[r4]], buf.at[lax.rem(j + 4, 5)], gsem)

        p = lax.rem(j, 5)
        pltpu.make_async_copy(g_hbm.at[srcv.at[0]], buf.at[p], gsem).wait()
        pltpu.make_async_copy(dsts_hbm.at[w, 0], dstv.at[0], dsem).wait()
        pltpu.sync_copy(buf.at[p], ysh.at[dstv.at[lax.rem(j, 6)]], add=True)
        return carry

    lax.fori_loop(0, KPT, body, 0)
    plsc.subcore_barrier()
    pltpu.sync_copy(ysh.at[pl.ds(s * RPS, RPS)], out_hbm.at[c, pl.ds(s * RPS, RPS)])


_BLK = 512


def _prep_body(d2_ref, x_ref, dv_ref, g_ref):
    deg = d2_ref[0][:, 0:1] + d2_ref[1][:, 0:1] + 1.0
    dinv = lax.rsqrt(deg)
    dvb = jnp.broadcast_to(dinv, x_ref.shape)
    dv_ref[...] = dvb
    g_ref[...] = dvb * x_ref[...]


def _tc_prep(deg2, xp):
    grid = (NP // _BLK,)
    return pl.pallas_call(
        _prep_body,
        grid=grid,
        in_specs=[
            pl.BlockSpec((2, _BLK, D), lambda i: (0, i, 0)),
            pl.BlockSpec((_BLK, D), lambda i: (i, 0)),
        ],
        out_specs=[
            pl.BlockSpec((_BLK, D), lambda i: (i, 0)),
            pl.BlockSpec((_BLK, D), lambda i: (i, 0)),
        ],
        out_shape=[
            jax.ShapeDtypeStruct((NP, D), jnp.float32),
            jax.ShapeDtypeStruct((NP, D), jnp.float32),
        ],
    )(deg2, xp)


def _layer_body(relu, want_g, a_ref, g_ref, dv_ref, w_ref, b_ref, *out_refs):
    z = dv_ref[...] * (a_ref[0] + a_ref[1] + g_ref[...])
    h = jnp.dot(z, w_ref[...], preferred_element_type=jnp.float32,
                precision=lax.Precision.HIGHEST) + b_ref[...]
    if relu:
        h = jnp.maximum(h, 0.0)
    out_refs[0][...] = h
    if want_g:
        out_refs[1][...] = dv_ref[...] * h


def _tc_layer(a2, g, dvb, W, b, relu, want_g):
    F = W.shape[1]
    grid = (NP // _BLK,)
    out_shape = [jax.ShapeDtypeStruct((NP, F), jnp.float32)]
    out_specs = [pl.BlockSpec((_BLK, F), lambda i: (i, 0))]
    if want_g:
        out_shape.append(jax.ShapeDtypeStruct((NP, F), jnp.float32))
        out_specs.append(pl.BlockSpec((_BLK, F), lambda i: (i, 0)))
    return pl.pallas_call(
        functools.partial(_layer_body, relu, want_g),
        grid=grid,
        in_specs=[
            pl.BlockSpec((2, _BLK, D), lambda i: (0, i, 0)),
            pl.BlockSpec((_BLK, D), lambda i: (i, 0)),
            pl.BlockSpec((_BLK, D), lambda i: (i, 0)),
            pl.BlockSpec((D, F), lambda i: (0, 0)),
            pl.BlockSpec((1, F), lambda i: (0, 0)),
        ],
        out_specs=out_specs,
        out_shape=out_shape,
    )(a2, g, dvb, W, b)


def kernel(x, edge_index, W1, b1, W2, b2, W3, b3):
    src = edge_index[0].astype(jnp.int32)
    dst = edge_index[1].astype(jnp.int32)
    pad = EP - E
    src = jnp.concatenate([src, jnp.zeros((pad,), jnp.int32)])
    dst = jnp.concatenate([dst, jnp.full((pad,), N, jnp.int32)])
    srcs = src.reshape(NW, KPT, C)
    dsts = dst.reshape(NW, KPT, C)

    xp = jnp.pad(x, ((0, NP - N), (0, 0)))
    onesD = jnp.ones((C, D), jnp.float32)
    zerosD = jnp.zeros((NP, D), jnp.float32)

    deg2 = _sc_degree(dsts, onesD, zerosD)
    dvb, g1 = _tc_prep(deg2, xp)

    a1 = _sc_aggregate(g1, srcs, dsts, zerosD)
    h1, g2 = _tc_layer(a1, g1, dvb, W1, b1.reshape(1, -1), True, True)

    a2 = _sc_aggregate(g2, srcs, dsts, zerosD)
    h2, g3 = _tc_layer(a2, g2, dvb, W2, b2.reshape(1, -1), True, True)

    a3 = _sc_aggregate(g3, srcs, dsts, zerosD)
    (out,) = _tc_layer(a3, g3, dvb, W3, b3.reshape(1, -1), False, False)

    return (h2[:N], out[:N])
